# baseline (speedup 1.0000x reference)
import functools

import jax
import jax.numpy as jnp
from jax import lax
from jax.experimental import pallas as pl
from jax.experimental.pallas import tpu as pltpu
from jax.experimental.pallas import tpu_sc as plsc

NC = 2
NS = 16
NW = NC * NS


@functools.lru_cache(maxsize=None)
def _build_gather(B: int, D: int):
    assert B % NW == 0
    bpw = B // NW
    C = 800
    assert bpw % C == 0
    nchunk = bpw // C

    mesh = plsc.VectorSubcoreMesh(
        core_axis_name="c", subcore_axis_name="s",
        num_cores=NC, num_subcores=NS)

    @functools.partial(
        pl.kernel,
        out_type=jax.ShapeDtypeStruct((B, D), jnp.float32),
        mesh=mesh,
        compiler_params=pltpu.CompilerParams(use_tc_tiling_on_sc=False),
        scratch_types=[
            pltpu.VMEM((C,), jnp.int32),
            pltpu.VMEM((C, D), jnp.float32),
            pltpu.SemaphoreType.DMA,
        ],
    )
    def gather_kernel(x_hbm, table_hbm, out_hbm, idx_v, rows_v, sem):
        wid = lax.axis_index("s") * NC + lax.axis_index("c")
        base = wid * bpw

        def body(j, carry):
            off = base + j * C
            pltpu.sync_copy(x_hbm.at[pl.ds(off, C)], idx_v)
            pltpu.async_copy(table_hbm.at[idx_v], rows_v, sem).wait()
            pltpu.sync_copy(rows_v, out_hbm.at[pl.ds(off, C)])
            return carry

        lax.fori_loop(0, nchunk, body, 0)

    return gather_kernel


def _widen_block(lo_ref, hi_ref, out_ref):
    out_ref[:, 0:64] = lo_ref[:]
    out_ref[:, 64:128] = hi_ref[:]


@functools.lru_cache(maxsize=None)
def _build_widen(V: int, D: int):
    CB = 4000
    H = V // 2
    assert H % CB == 0
    grid = H // CB
    return pl.pallas_call(
        _widen_block,
        grid=(grid,),
        in_specs=[
            pl.BlockSpec((CB, D), lambda i: (i, 0)),
            pl.BlockSpec((CB, D), lambda i, g=grid: (i + g, 0)),
        ],
        out_specs=pl.BlockSpec((CB, 2 * D), lambda i: (i, 0)),
        out_shape=jax.ShapeDtypeStruct((H, 2 * D), jnp.float32),
    )


def kernel(x, table):
    B = x.shape[0] * x.shape[1]
    V, D = table.shape
    H = V // 2
    t128 = _build_widen(V, D)(table, table)
    tlin = t128.reshape(V, D)
    xi = x.reshape(B)
    idx = (xi % H) * 2 + xi // H
    out = _build_gather(B, D)(idx, tlin)
    return out.reshape(x.shape[0], x.shape[1], D)


# speedup vs baseline: 1.7514x; 1.7514x over previous
import functools

import jax
import jax.numpy as jnp
from jax import lax
from jax.experimental import pallas as pl
from jax.experimental.pallas import tpu as pltpu
from jax.experimental.pallas import tpu_sc as plsc

NC = 2
NS = 16
NW = NC * NS

WB = 8192


@functools.lru_cache(maxsize=None)
def _build_gather(B: int, D: int):
    assert B % NW == 0
    bpw = B // NW
    C = 800
    assert bpw % C == 0
    nchunk = bpw // C

    mesh = plsc.VectorSubcoreMesh(
        core_axis_name="c", subcore_axis_name="s",
        num_cores=NC, num_subcores=NS)

    @functools.partial(
        pl.kernel,
        out_type=jax.ShapeDtypeStruct((B, D), jnp.float32),
        mesh=mesh,
        compiler_params=pltpu.CompilerParams(use_tc_tiling_on_sc=False),
        scratch_types=[
            pltpu.VMEM((C,), jnp.int32),
            pltpu.VMEM((C, D), jnp.float32),
            pltpu.SemaphoreType.DMA,
        ],
    )
    def gather_kernel(x_hbm, table_hbm, out_hbm, idx_v, rows_v, sem):
        wid = lax.axis_index("s") * NC + lax.axis_index("c")
        base = wid * bpw

        def body(j, carry):
            off = base + j * C
            pltpu.sync_copy(x_hbm.at[pl.ds(off, C)], idx_v)
            pltpu.async_copy(table_hbm.at[idx_v], rows_v, sem).wait()
            pltpu.sync_copy(rows_v, out_hbm.at[pl.ds(off, C)])
            return carry

        lax.fori_loop(0, nchunk, body, 0)

    return gather_kernel


def _widen_block(in_ref, out_ref):
    x = in_ref[:]
    h = WB // 2
    out_ref[:, 0:64] = x[:, 0:h].T
    out_ref[:, 64:128] = x[:, h:].T


@functools.lru_cache(maxsize=None)
def _build_widen(V: int, D: int):
    grid = (V + WB - 1) // WB
    return pl.pallas_call(
        _widen_block,
        grid=(grid,),
        in_specs=[pl.BlockSpec((D, WB), lambda i: (0, i))],
        out_specs=pl.BlockSpec((WB // 2, 2 * D), lambda i: (i, 0)),
        out_shape=jax.ShapeDtypeStruct((grid * (WB // 2), 2 * D), jnp.float32),
    )


def kernel(x, table):
    B = x.shape[0] * x.shape[1]
    V, D = table.shape
    t128 = _build_widen(V, D)(table.T)
    tlin = t128.reshape(t128.shape[0] * 2, D)
    xi = x.reshape(B)
    h = WB // 2
    idx = (xi // WB) * WB + 2 * (xi % h) + (xi % WB) // h
    out = _build_gather(B, D)(idx, tlin)
    return out.reshape(x.shape[0], x.shape[1], D)


# speedup vs baseline: 1.8821x; 1.0746x over previous
import functools

import jax
import jax.numpy as jnp
from jax import lax
from jax.experimental import pallas as pl
from jax.experimental.pallas import tpu as pltpu
from jax.experimental.pallas import tpu_sc as plsc

NC = 2
NS = 16
NW = NC * NS

WB = 16384


@functools.lru_cache(maxsize=None)
def _build_gather(B: int, D: int):
    assert B % NW == 0
    bpw = B // NW
    C = 800
    assert bpw % C == 0
    nchunk = bpw // C

    mesh = plsc.VectorSubcoreMesh(
        core_axis_name="c", subcore_axis_name="s",
        num_cores=NC, num_subcores=NS)

    @functools.partial(
        pl.kernel,
        out_type=jax.ShapeDtypeStruct((B, D), jnp.float32),
        mesh=mesh,
        compiler_params=pltpu.CompilerParams(use_tc_tiling_on_sc=False),
        scratch_types=[
            pltpu.VMEM((C,), jnp.int32),
            pltpu.VMEM((C, D), jnp.float32),
            pltpu.SemaphoreType.DMA,
        ],
    )
    def gather_kernel(x_hbm, table_hbm, out_hbm, idx_v, rows_v, sem):
        wid = lax.axis_index("s") * NC + lax.axis_index("c")
        base = wid * bpw

        def body(j, carry):
            off = base + j * C
            pltpu.sync_copy(x_hbm.at[pl.ds(off, C)], idx_v)
            pltpu.async_copy(table_hbm.at[idx_v], rows_v, sem).wait()
            pltpu.sync_copy(rows_v, out_hbm.at[pl.ds(off, C)])
            return carry

        lax.fori_loop(0, nchunk, body, 0)

    return gather_kernel


def _widen_block(in_ref, out_ref):
    x = in_ref[:]
    h = WB // 2
    out_ref[:] = jnp.concatenate([x[:, 0:h].T, x[:, h:].T], axis=1)


@functools.lru_cache(maxsize=None)
def _build_widen(V: int, D: int):
    grid = (V + WB - 1) // WB
    return pl.pallas_call(
        _widen_block,
        grid=(grid,),
        in_specs=[pl.BlockSpec((D, WB), lambda i: (0, i))],
        out_specs=pl.BlockSpec((WB // 2, 2 * D), lambda i: (i, 0)),
        out_shape=jax.ShapeDtypeStruct((grid * (WB // 2), 2 * D), jnp.float32),
    )


def kernel(x, table):
    B = x.shape[0] * x.shape[1]
    V, D = table.shape
    t128 = _build_widen(V, D)(table.T)
    tlin = t128.reshape(t128.shape[0] * 2, D)
    xi = x.reshape(B)
    h = WB // 2
    idx = (xi // WB) * WB + 2 * (xi % h) + (xi % WB) // h
    out = _build_gather(B, D)(idx, tlin)
    o128 = lax.optimization_barrier(out.reshape(B // 2, 2 * D))
    return o128.reshape(x.shape[0], x.shape[1], D)


# speedup vs baseline: 1.9572x; 1.0399x over previous
import functools

import jax
import jax.numpy as jnp
from jax import lax
from jax.experimental import pallas as pl
from jax.experimental.pallas import tpu as pltpu
from jax.experimental.pallas import tpu_sc as plsc

NC = 2
NS = 16
NW = NC * NS

WB = 16384


@functools.lru_cache(maxsize=None)
def _build_gather(B: int, D: int):
    assert B % NW == 0
    bpw = B // NW
    C = 800
    assert bpw % C == 0
    nchunk = bpw // C

    mesh = plsc.VectorSubcoreMesh(
        core_axis_name="c", subcore_axis_name="s",
        num_cores=NC, num_subcores=NS)

    @functools.partial(
        pl.kernel,
        out_type=jax.ShapeDtypeStruct((B, D), jnp.float32),
        mesh=mesh,
        compiler_params=pltpu.CompilerParams(use_tc_tiling_on_sc=False),
        scratch_types=[
            pltpu.VMEM((C,), jnp.int32),
            pltpu.VMEM((C, D), jnp.float32),
            pltpu.SemaphoreType.DMA,
        ],
    )
    def gather_kernel(x_hbm, table_hbm, out_hbm, idx_v, rows_v, sem):
        wid = lax.axis_index("s") * NC + lax.axis_index("c")
        base = wid * bpw

        def body(j, carry):
            off = base + j * C
            pltpu.sync_copy(x_hbm.at[pl.ds(off, C)], idx_v)
            pltpu.async_copy(table_hbm.at[idx_v], rows_v, sem).wait()
            pltpu.sync_copy(rows_v, out_hbm.at[pl.ds(off, C)])
            return carry

        lax.fori_loop(0, nchunk, body, 0)

    return gather_kernel


def _widen_block(in_ref, out_ref):
    x = in_ref[:]
    h = WB // 2
    out_ref[:] = jnp.concatenate([x[:, 0:h].T, x[:, h:].T], axis=1)


@functools.lru_cache(maxsize=None)
def _build_widen(V: int, D: int):
    grid = (V + WB - 1) // WB
    return pl.pallas_call(
        _widen_block,
        grid=(grid,),
        in_specs=[pl.BlockSpec((D, WB), lambda i: (0, i))],
        out_specs=pl.BlockSpec((WB // 2, 2 * D), lambda i: (i, 0)),
        out_shape=jax.ShapeDtypeStruct((grid * (WB // 2), 2 * D), jnp.float32),
    )


def _format_block(in_ref, out_ref):
    z = in_ref[:].T
    hb = out_ref.shape[1] // 2
    out_ref[:, 0:hb] = z[0:64, :]
    out_ref[:, hb:] = z[64:128, :]


@functools.lru_cache(maxsize=None)
def _build_format(BATCH: int, HIST: int, D: int):
    return pl.pallas_call(
        _format_block,
        grid=(HIST,),
        in_specs=[pl.BlockSpec((BATCH // 2, 2 * D), lambda i: (i, 0))],
        out_specs=pl.BlockSpec((D, BATCH), lambda i: (i, 0)),
        out_shape=jax.ShapeDtypeStruct((HIST * D, BATCH), jnp.float32),
    )


def kernel(x, table):
    BATCH, HIST = x.shape
    B = BATCH * HIST
    V, D = table.shape
    t128 = _build_widen(V, D)(table.T)
    tlin = t128.reshape(t128.shape[0] * 2, D)
    hb = BATCH // 2
    xi = x.T.reshape(HIST, 2, hb).transpose(0, 2, 1).reshape(B)
    h = WB // 2
    idx = (xi // WB) * WB + 2 * (xi % h) + (xi % WB) // h
    out = _build_gather(B, D)(idx, tlin)
    outf = _build_format(BATCH, HIST, D)(out.reshape(B // 2, 2 * D))
    return outf.reshape(HIST, D, BATCH).transpose(2, 0, 1)
